# baseline (speedup 1.0000x reference)
import functools

import jax
import jax.numpy as jnp
from jax import lax
from jax.experimental import pallas as pl
from jax.experimental.pallas import tpu as pltpu
from jax.experimental.pallas import tpu_sc as plsc

B = 4
N = 4096
NC = 2
NS = 16
NW = NC * NS
SC_ROWS = 4096
SC_CHUNK = SC_ROWS // NW
TC_SKIP = 2
IB = 32
NT = IB // 16
SUBACC = 2
LANES = 16
SLICE = N // NS
TN = 2048


def _sc_body(
    x1x, x1y, x1z, x2x, x2y, x2z, rowparts, colparts,
    qx, qy, qz, rx, ry, rz, colacc, dtile, idxcols, redbuf, mslice, ovec, shared,
):
    c = lax.axis_index("c")
    s = lax.axis_index("s")
    wid = s * 2 + c
    base = wid * SC_CHUNK

    iota = lax.iota(jnp.int32, LANES)
    for t in range(NT):
        for i in range(LANES):
            idxcols[pl.ds((t * LANES + i) * LANES, LANES)] = (
                iota * LANES + (t * LANES * LANES + i)
            )

    inf = jnp.full((LANES,), jnp.inf, jnp.float32)

    def initbody(v, _):
        colacc[pl.ds(v * LANES, LANES)] = inf
        return 0

    lax.fori_loop(0, N // LANES, initbody, 0)

    for src, dst in zip((x1x, x1y, x1z), (qx, qy, qz)):
        pltpu.sync_copy(src.at[pl.ds(base, SC_CHUNK)], dst)
    for src, dst in zip((x2x, x2y, x2z), (rx, ry, rz)):
        pltpu.sync_copy(src, dst)

    def ibody(ib, vtotal):
        qxv = [qx[pl.ds(ib * IB + t * LANES, LANES)] for t in range(NT)]
        qyv = [qy[pl.ds(ib * IB + t * LANES, LANES)] for t in range(NT)]
        qzv = [qz[pl.ds(ib * IB + t * LANES, LANES)] for t in range(NT)]

        def jbody(jv, accs, qxv=qxv, qyv=qyv, qzv=qzv):
            rxv = rx[pl.ds(jv * LANES, LANES)]
            ryv = ry[pl.ds(jv * LANES, LANES)]
            rzv = rz[pl.ds(jv * LANES, LANES)]
            accs = list(accs)
            for l in range(LANES):
                sx = rxv[l]
                sy = ryv[l]
                sz = rzv[l]
                for t in range(NT):
                    dx = qxv[t] - sx
                    dy = qyv[t] - sy
                    dz = qzv[t] - sz
                    d = dx * dx + dy * dy + dz * dz
                    k = (l % SUBACC) * NT + t
                    accs[k] = jnp.minimum(accs[k], d)
                    dtile[pl.ds((t * LANES + l) * LANES, LANES)] = d
            g = []
            for t in range(NT):
                for i in range(LANES):
                    idxv = idxcols[pl.ds((t * LANES + i) * LANES, LANES)]
                    g.append(plsc.load_gather(dtile, [idxv]))
            while len(g) > 1:
                g = [jnp.minimum(g[2 * k], g[2 * k + 1]) for k in range(len(g) // 2)]
            cv = colacc[pl.ds(jv * LANES, LANES)]
            colacc[pl.ds(jv * LANES, LANES)] = jnp.minimum(cv, g[0])
            return tuple(accs)

        accs = lax.fori_loop(0, N // LANES, jbody, (inf,) * (SUBACC * NT))
        blocksum = None
        for t in range(NT):
            m = accs[t]
            for k in range(1, SUBACC):
                m = jnp.minimum(m, accs[k * NT + t])
            blocksum = m if blocksum is None else blocksum + m
        return vtotal + blocksum

    vtotal = lax.fori_loop(0, SC_CHUNK // IB, ibody, jnp.zeros((LANES,), jnp.float32))
    ovec[...] = vtotal * jnp.float32(1.0 / N)
    pltpu.sync_copy(ovec, rowparts.at[wid])

    pltpu.sync_copy(colacc, shared.at[s])
    plsc.subcore_barrier()
    pltpu.sync_copy(shared.at[pl.ds(0, NS), pl.ds(s * SLICE, SLICE)], redbuf)

    def redbody(v, _):
        m = redbuf[0, pl.ds(v * LANES, LANES)]
        for r in range(1, NS):
            m = jnp.minimum(m, redbuf[r, pl.ds(v * LANES, LANES)])
        mslice[pl.ds(v * LANES, LANES)] = m
        return 0

    lax.fori_loop(0, SLICE // LANES, redbody, 0)
    pltpu.sync_copy(mslice, colparts.at[c, pl.ds(s * SLICE, SLICE)])


def _tc_main_body(x1_ref, x2_ref, colmin_ref, rowagg_ref):
    k = pl.program_id(0)
    i = (k + TC_SKIP) % (N // TN)
    a = x1_ref[0]
    b = x2_ref[0]
    qn = a[0] * a[0] + a[1] * a[1] + a[2] * a[2]
    rn = b[0] * b[0] + b[1] * b[1] + b[2] * b[2]
    am2 = a * jnp.float32(-2.0)
    lhs = jnp.concatenate(
        [am2, qn[None, :], jnp.ones((1, TN), jnp.float32)], axis=0
    )
    rhs = jnp.concatenate(
        [b, jnp.ones((1, N), jnp.float32), rn[None, :]], axis=0
    )
    d = lax.dot_general(
        lhs, rhs, (((0,), (0,)), ((), ())),
        precision=lax.Precision.HIGHEST,
        preferred_element_type=jnp.float32,
    )
    rowpart = jnp.sum(jnp.min(d, axis=1)) * jnp.float32(1.0 / N)
    cmin = jnp.min(d, axis=0, keepdims=True)

    first = jnp.logical_or(i == 0, k == 0)

    @pl.when(first)
    def _():
        rowagg_ref[...] = jnp.full((1, 1, 128), rowpart, jnp.float32)
        colmin_ref[...] = cmin[None]

    @pl.when(jnp.logical_not(first))
    def _():
        rowagg_ref[...] = rowagg_ref[...] + rowpart
        colmin_ref[...] = jnp.minimum(colmin_ref[...], cmin[None])


def _tc_merge_body(colmin_ref, rowagg_ref, sccol_ref, scrow_ref, out_ref):
    colmean = jnp.mean(colmin_ref[:, 0, :], axis=1)
    tc_out = rowagg_ref[:, 0, 0] + colmean
    sc_col = jnp.minimum(sccol_ref[0, :], sccol_ref[1, :])
    sc_out = jnp.sum(scrow_ref[...])
    if TC_SKIP < N // TN:
        sc_col = jnp.minimum(sc_col, colmin_ref[0, 0, :])
        sc_out = sc_out + rowagg_ref[0, 0, 0]
    sc_out = sc_out + jnp.mean(sc_col)
    res = jnp.where(lax.iota(jnp.int32, B) == 0, sc_out, tc_out)
    out_ref[...] = jnp.broadcast_to(res[:, None], (B, 128))


def kernel(xyz1, xyz2):
    x1 = jnp.transpose(xyz1, (2, 0, 1))
    x2 = jnp.transpose(xyz2, (2, 0, 1))
    x1t = jnp.transpose(xyz1, (0, 2, 1))
    x2t = jnp.transpose(xyz2, (0, 2, 1))

    sc_run = functools.partial(
        pl.kernel,
        mesh=plsc.VectorSubcoreMesh(core_axis_name="c", subcore_axis_name="s"),
        compiler_params=pltpu.CompilerParams(needs_layout_passes=False),
        out_type=(
            jax.ShapeDtypeStruct((NW, LANES), jnp.float32),
            jax.ShapeDtypeStruct((NC, N), jnp.float32),
        ),
        scratch_types=[
            pltpu.VMEM((SC_CHUNK,), jnp.float32),
            pltpu.VMEM((SC_CHUNK,), jnp.float32),
            pltpu.VMEM((SC_CHUNK,), jnp.float32),
            pltpu.VMEM((N,), jnp.float32),
            pltpu.VMEM((N,), jnp.float32),
            pltpu.VMEM((N,), jnp.float32),
            pltpu.VMEM((N,), jnp.float32),
            pltpu.VMEM((NT * LANES * LANES,), jnp.float32),
            pltpu.VMEM((NT * LANES * LANES,), jnp.int32),
            pltpu.VMEM((NS, SLICE), jnp.float32),
            pltpu.VMEM((SLICE,), jnp.float32),
            pltpu.VMEM((LANES,), jnp.float32),
            pltpu.VMEM_SHARED((NS, N), jnp.float32),
        ],
    )(_sc_body)
    rowparts, colparts = sc_run(
        x1[0][0], x1[1][0], x1[2][0], x2[0][0], x2[1][0], x2[2][0]
    )

    ntiles = N // TN
    colmin, rowagg = pl.pallas_call(
        _tc_main_body,
        grid=(B * ntiles - TC_SKIP,),
        in_specs=[
            pl.BlockSpec(
                (1, 3, TN), lambda k: ((k + TC_SKIP) // ntiles, 0, (k + TC_SKIP) % ntiles)
            ),
            pl.BlockSpec((1, 3, N), lambda k: ((k + TC_SKIP) // ntiles, 0, 0)),
        ],
        out_specs=[
            pl.BlockSpec((1, 1, N), lambda k: ((k + TC_SKIP) // ntiles, 0, 0)),
            pl.BlockSpec((1, 1, 128), lambda k: ((k + TC_SKIP) // ntiles, 0, 0)),
        ],
        out_shape=[
            jax.ShapeDtypeStruct((B, 1, N), jnp.float32),
            jax.ShapeDtypeStruct((B, 1, 128), jnp.float32),
        ],
    )(x1t, x2t)

    out = pl.pallas_call(
        _tc_merge_body,
        out_shape=jax.ShapeDtypeStruct((B, 128), jnp.float32),
    )(colmin, rowagg, colparts, rowparts)
    return out[:, 0]


# speedup vs baseline: 1.0101x; 1.0101x over previous
import functools

import jax
import jax.numpy as jnp
from jax import lax
from jax.experimental import pallas as pl
from jax.experimental.pallas import tpu as pltpu
from jax.experimental.pallas import tpu_sc as plsc

B = 4
N = 4096
NC = 2
NS = 16
NW = NC * NS
SC_ROWS = 4096
SC_CHUNK = SC_ROWS // NW
TC_SKIP = 4
IB = 32
NT = IB // 16
SUBACC = 2
LANES = 16
SLICE = N // NS
TN = 1024


def _sc_body(
    x1x, x1y, x1z, x2x, x2y, x2z, rowparts, colparts,
    qx, qy, qz, rx, ry, rz, colacc, dtile, idxcols, redbuf, mslice, ovec, shared,
):
    c = lax.axis_index("c")
    s = lax.axis_index("s")
    wid = s * 2 + c
    base = wid * SC_CHUNK

    iota = lax.iota(jnp.int32, LANES)
    for h in range(2):
        for t in range(NT):
            for i in range(LANES):
                idxcols[pl.ds(((h * NT + t) * LANES + i) * LANES, LANES)] = (
                    iota * LANES + ((h * NT + t) * LANES * LANES + i)
                )

    inf = jnp.full((LANES,), jnp.inf, jnp.float32)

    def initbody(v, _):
        colacc[pl.ds(v * LANES, LANES)] = inf
        return 0

    lax.fori_loop(0, N // LANES, initbody, 0)

    for src, dst in zip((x1x, x1y, x1z), (qx, qy, qz)):
        pltpu.sync_copy(src.at[pl.ds(base, SC_CHUNK)], dst)
    for src, dst in zip((x2x, x2y, x2z), (rx, ry, rz)):
        pltpu.sync_copy(src, dst)

    def ibody(ib, vtotal):
        qxv = [qx[pl.ds(ib * IB + t * LANES, LANES)] for t in range(NT)]
        qyv = [qy[pl.ds(ib * IB + t * LANES, LANES)] for t in range(NT)]
        qzv = [qz[pl.ds(ib * IB + t * LANES, LANES)] for t in range(NT)]

        def compute_tile(jv, h, accs):
            rxv = rx[pl.ds(jv * LANES, LANES)]
            ryv = ry[pl.ds(jv * LANES, LANES)]
            rzv = rz[pl.ds(jv * LANES, LANES)]
            for l in range(LANES):
                sx = rxv[l]
                sy = ryv[l]
                sz = rzv[l]
                for t in range(NT):
                    dx = qxv[t] - sx
                    dy = qyv[t] - sy
                    dz = qzv[t] - sz
                    d = dx * dx + dy * dy + dz * dz
                    k = (l % SUBACC) * NT + t
                    accs[k] = jnp.minimum(accs[k], d)
                    dtile[pl.ds(((h * NT + t) * LANES + l) * LANES, LANES)] = d
            return accs

        def gather_tile(jv, h):
            g = []
            for t in range(NT):
                for i in range(LANES):
                    idxv = idxcols[pl.ds(((h * NT + t) * LANES + i) * LANES, LANES)]
                    g.append(plsc.load_gather(dtile, [idxv]))
            while len(g) > 1:
                g = [jnp.minimum(g[2 * k], g[2 * k + 1]) for k in range(len(g) // 2)]
            cv = colacc[pl.ds(jv * LANES, LANES)]
            colacc[pl.ds(jv * LANES, LANES)] = jnp.minimum(cv, g[0])

        def jbody(jv, accs):
            accs = list(accs)
            accs = compute_tile(jv, 0, accs)
            gather_tile(jv, 0)
            return tuple(accs)

        accs = lax.fori_loop(0, N // LANES, jbody, (inf,) * (SUBACC * NT))
        blocksum = None
        for t in range(NT):
            m = accs[t]
            for k in range(1, SUBACC):
                m = jnp.minimum(m, accs[k * NT + t])
            blocksum = m if blocksum is None else blocksum + m
        return vtotal + blocksum

    vtotal = lax.fori_loop(0, SC_CHUNK // IB, ibody, jnp.zeros((LANES,), jnp.float32))
    ovec[...] = vtotal * jnp.float32(1.0 / N)
    pltpu.sync_copy(ovec, rowparts.at[wid])

    pltpu.sync_copy(colacc, shared.at[s])
    plsc.subcore_barrier()
    pltpu.sync_copy(shared.at[pl.ds(0, NS), pl.ds(s * SLICE, SLICE)], redbuf)

    def redbody(v, _):
        m = redbuf[0, pl.ds(v * LANES, LANES)]
        for r in range(1, NS):
            m = jnp.minimum(m, redbuf[r, pl.ds(v * LANES, LANES)])
        mslice[pl.ds(v * LANES, LANES)] = m
        return 0

    lax.fori_loop(0, SLICE // LANES, redbody, 0)
    pltpu.sync_copy(mslice, colparts.at[c, pl.ds(s * SLICE, SLICE)])


def _tc_main_body(x1_ref, x2_ref, colmin_ref, rowagg_ref):
    k = pl.program_id(0)
    i = (k + TC_SKIP) % (N // TN)
    a = x1_ref[0]
    b = x2_ref[0]
    qn = a[0] * a[0] + a[1] * a[1] + a[2] * a[2]
    rn = b[0] * b[0] + b[1] * b[1] + b[2] * b[2]
    am2 = a * jnp.float32(-2.0)
    lhs = jnp.concatenate(
        [am2, qn[None, :], jnp.ones((1, TN), jnp.float32)], axis=0
    )
    rhs = jnp.concatenate(
        [b, jnp.ones((1, N), jnp.float32), rn[None, :]], axis=0
    )
    d = lax.dot_general(
        lhs, rhs, (((0,), (0,)), ((), ())),
        precision=lax.Precision.HIGHEST,
        preferred_element_type=jnp.float32,
    )
    rowpart = jnp.sum(jnp.min(d, axis=1)) * jnp.float32(1.0 / N)
    cmin = jnp.min(d, axis=0, keepdims=True)

    first = jnp.logical_or(i == 0, k == 0)

    @pl.when(first)
    def _():
        rowagg_ref[...] = jnp.full((1, 1, 128), rowpart, jnp.float32)
        colmin_ref[...] = cmin[None]

    @pl.when(jnp.logical_not(first))
    def _():
        rowagg_ref[...] = rowagg_ref[...] + rowpart
        colmin_ref[...] = jnp.minimum(colmin_ref[...], cmin[None])


def _tc_merge_body(colmin_ref, rowagg_ref, sccol_ref, scrow_ref, out_ref):
    colmean = jnp.mean(colmin_ref[:, 0, :], axis=1)
    tc_out = rowagg_ref[:, 0, 0] + colmean
    sc_col = jnp.minimum(sccol_ref[0, :], sccol_ref[1, :])
    sc_out = jnp.sum(scrow_ref[...])
    if TC_SKIP < N // TN:
        sc_col = jnp.minimum(sc_col, colmin_ref[0, 0, :])
        sc_out = sc_out + rowagg_ref[0, 0, 0]
    sc_out = sc_out + jnp.mean(sc_col)
    res = jnp.where(lax.iota(jnp.int32, B) == 0, sc_out, tc_out)
    out_ref[...] = jnp.broadcast_to(res[:, None], (B, 128))


def kernel(xyz1, xyz2):
    x1 = jnp.transpose(xyz1, (2, 0, 1))
    x2 = jnp.transpose(xyz2, (2, 0, 1))
    x1t = jnp.transpose(xyz1, (0, 2, 1))
    x2t = jnp.transpose(xyz2, (0, 2, 1))

    sc_run = functools.partial(
        pl.kernel,
        mesh=plsc.VectorSubcoreMesh(core_axis_name="c", subcore_axis_name="s"),
        compiler_params=pltpu.CompilerParams(needs_layout_passes=False),
        out_type=(
            jax.ShapeDtypeStruct((NW, LANES), jnp.float32),
            jax.ShapeDtypeStruct((NC, N), jnp.float32),
        ),
        scratch_types=[
            pltpu.VMEM((SC_CHUNK,), jnp.float32),
            pltpu.VMEM((SC_CHUNK,), jnp.float32),
            pltpu.VMEM((SC_CHUNK,), jnp.float32),
            pltpu.VMEM((N,), jnp.float32),
            pltpu.VMEM((N,), jnp.float32),
            pltpu.VMEM((N,), jnp.float32),
            pltpu.VMEM((N,), jnp.float32),
            pltpu.VMEM((2 * NT * LANES * LANES,), jnp.float32),
            pltpu.VMEM((2 * NT * LANES * LANES,), jnp.int32),
            pltpu.VMEM((NS, SLICE), jnp.float32),
            pltpu.VMEM((SLICE,), jnp.float32),
            pltpu.VMEM((LANES,), jnp.float32),
            pltpu.VMEM_SHARED((NS, N), jnp.float32),
        ],
    )(_sc_body)
    rowparts, colparts = sc_run(
        x1[0][0], x1[1][0], x1[2][0], x2[0][0], x2[1][0], x2[2][0]
    )

    ntiles = N // TN
    colmin, rowagg = pl.pallas_call(
        _tc_main_body,
        grid=(B * ntiles - TC_SKIP,),
        in_specs=[
            pl.BlockSpec(
                (1, 3, TN), lambda k: ((k + TC_SKIP) // ntiles, 0, (k + TC_SKIP) % ntiles)
            ),
            pl.BlockSpec((1, 3, N), lambda k: ((k + TC_SKIP) // ntiles, 0, 0)),
        ],
        out_specs=[
            pl.BlockSpec((1, 1, N), lambda k: ((k + TC_SKIP) // ntiles, 0, 0)),
            pl.BlockSpec((1, 1, 128), lambda k: ((k + TC_SKIP) // ntiles, 0, 0)),
        ],
        out_shape=[
            jax.ShapeDtypeStruct((B, 1, N), jnp.float32),
            jax.ShapeDtypeStruct((B, 1, 128), jnp.float32),
        ],
    )(x1t, x2t)

    out = pl.pallas_call(
        _tc_merge_body,
        out_shape=jax.ShapeDtypeStruct((B, 128), jnp.float32),
    )(colmin, rowagg, colparts, rowparts)
    return out[:, 0]
